# baseline (speedup 1.0000x reference)
import functools

import jax
import jax.numpy as jnp
from jax import lax
from jax.experimental import pallas as pl
from jax.experimental.pallas import tpu as pltpu
from jax.experimental.pallas import tpu_sc as plsc

NUM_TASKS = 8
NUM_TOKENS = 4096
INPUT_DIM = 1024
NUM_CLASSES = 512

TB = 256
NB = NUM_TOKENS // TB + NUM_TASKS
P = NB * TB

_NW = 32


def _sc_mesh():
    return plsc.VectorSubcoreMesh(core_axis_name="c", subcore_axis_name="s")


_G_ROWS = P // _NW
_G_CH = _G_ROWS // 2


def _gather_feat_kernel(feat_hbm, idx_hbm, out_hbm, idx_v, rows_v, sem):
    wid = lax.axis_index("s") * 2 + lax.axis_index("c")
    base = wid * _G_ROWS
    pltpu.sync_copy(idx_hbm.at[pl.ds(base, _G_ROWS)], idx_v)
    for c in range(_G_ROWS // _G_CH):
        pltpu.async_copy(
            feat_hbm.at[idx_v.at[pl.ds(c * _G_CH, _G_CH)]], rows_v, sem
        ).wait()
        pltpu.sync_copy(rows_v, out_hbm.at[pl.ds(base + c * _G_CH, _G_CH)])


def _gather_feat(feature, row_token):
    return pl.kernel(
        _gather_feat_kernel,
        mesh=_sc_mesh(),
        out_type=jax.ShapeDtypeStruct((P, INPUT_DIM), jnp.float32),
        scratch_types=[
            pltpu.VMEM((_G_ROWS,), jnp.int32),
            pltpu.VMEM((_G_CH, INPUT_DIM), jnp.float32),
            pltpu.SemaphoreType.DMA,
        ],
    )(feature, row_token)


_C_ROWS = NUM_TOKENS // _NW


def _combine_kernel(slog_hbm, idx_hbm, out_hbm, idx_v, rows_v, sem):
    wid = lax.axis_index("s") * 2 + lax.axis_index("c")
    base = wid * _C_ROWS
    pltpu.sync_copy(idx_hbm.at[pl.ds(base, _C_ROWS)], idx_v)
    pltpu.async_copy(slog_hbm.at[idx_v], rows_v, sem).wait()
    pltpu.sync_copy(rows_v, out_hbm.at[pl.ds(base, _C_ROWS)])


def _combine(sorted_logits, p):
    return pl.kernel(
        _combine_kernel,
        mesh=_sc_mesh(),
        out_type=jax.ShapeDtypeStruct((NUM_TOKENS, NUM_CLASSES), jnp.float32),
        scratch_types=[
            pltpu.VMEM((_C_ROWS,), jnp.int32),
            pltpu.VMEM((_C_ROWS, NUM_CLASSES), jnp.float32),
            pltpu.SemaphoreType.DMA,
        ],
    )(sorted_logits, p)


def _mm_body(nu_ref, bt_ref, x_ref, w_ref, b_ref, o_ref):
    i = pl.program_id(0)

    @pl.when(i < nu_ref[0])
    def _():
        x = x_ref[...]
        w = w_ref[0]
        y = lax.dot_general(x, w, (((1,), (1,)), ((), ())),
                            preferred_element_type=jnp.float32)
        o_ref[...] = y + b_ref[0, 0][None, :]


def _grouped_matmul(sorted_feat, W, b, nb_used, block_task):
    grid_spec = pltpu.PrefetchScalarGridSpec(
        num_scalar_prefetch=2,
        grid=(NB,),
        in_specs=[
            pl.BlockSpec((TB, INPUT_DIM), lambda i, nu, bt: (i, 0)),
            pl.BlockSpec((1, NUM_CLASSES, INPUT_DIM),
                         lambda i, nu, bt: (bt[i], 0, 0)),
            pl.BlockSpec((1, 1, NUM_CLASSES), lambda i, nu, bt: (bt[i], 0, 0)),
        ],
        out_specs=pl.BlockSpec((TB, NUM_CLASSES), lambda i, nu, bt: (i, 0)),
    )
    return pl.pallas_call(
        _mm_body,
        grid_spec=grid_spec,
        out_shape=jax.ShapeDtypeStruct((P, NUM_CLASSES), jnp.float32),
    )(nb_used, block_task, sorted_feat, W,
      b.reshape(NUM_TASKS, 1, NUM_CLASSES))


def kernel(feature, task_ids, W, b):
    t = task_ids.astype(jnp.int32)
    onehot = (t[:, None] == jnp.arange(NUM_TASKS, dtype=jnp.int32)[None, :])
    csum = jnp.cumsum(onehot.astype(jnp.int32), axis=0)
    counts = csum[-1]
    rank = jnp.take_along_axis(csum, t[:, None], axis=1)[:, 0] - 1
    blocks_per = (counts + TB - 1) // TB
    cumb = jnp.cumsum(blocks_per)
    padded_off = TB * jnp.concatenate(
        [jnp.zeros((1,), jnp.int32), cumb[:-1]])
    p = (padded_off[t] + rank).astype(jnp.int32)
    row_token = jnp.zeros((P,), jnp.int32).at[p].set(
        jnp.arange(NUM_TOKENS, dtype=jnp.int32))
    nb_used = cumb[-1:].astype(jnp.int32)
    block_task = jnp.minimum(
        jnp.searchsorted(cumb, jnp.arange(NB, dtype=jnp.int32),
                         side="right"),
        NUM_TASKS - 1).astype(jnp.int32)

    sorted_feat = _gather_feat(feature, row_token)
    sorted_logits = _grouped_matmul(sorted_feat, W, b, nb_used, block_task)
    return _combine(sorted_logits, p)


# speedup vs baseline: 1.3667x; 1.3667x over previous
import functools

import jax
import jax.numpy as jnp
from jax import lax
from jax.experimental import pallas as pl
from jax.experimental.pallas import tpu as pltpu
from jax.experimental.pallas import tpu_sc as plsc

NUM_TASKS = 8
NUM_TOKENS = 4096
INPUT_DIM = 1024
NUM_CLASSES = 512

TB = 256
NB = NUM_TOKENS // TB + NUM_TASKS
P = NB * TB

_NW = 32


def _sc_mesh():
    return plsc.VectorSubcoreMesh(core_axis_name="c", subcore_axis_name="s")


_G_W = 512
_G_SPLIT = INPUT_DIM // _G_W
_G_ROWS = P * _G_SPLIT // _NW
_G_CH = 96
_G_NCH = _G_ROWS // _G_CH


def _gather_feat_kernel(feat_hbm, idx_hbm, out_hbm, i0, i1, i2, i3,
                        rows0, rows1, sem0, sem1):
    wid = lax.axis_index("s") * 2 + lax.axis_index("c")
    base = wid * _G_ROWS
    idxs = (i0, i1, i2, i3)
    for c in range(_G_NCH):
        pltpu.sync_copy(idx_hbm.at[pl.ds(base + c * _G_CH, _G_CH)], idxs[c])
    bufs, sems, cps = (rows0, rows1), (sem0, sem1), [None, None]
    cps[0] = pltpu.async_copy(feat_hbm.at[i0], rows0, sem0)
    for c in range(_G_NCH):
        nxt = c + 1
        if nxt < _G_NCH:
            cps[nxt % 2] = pltpu.async_copy(
                feat_hbm.at[idxs[nxt]], bufs[nxt % 2], sems[nxt % 2])
        cps[c % 2].wait()
        pltpu.sync_copy(bufs[c % 2],
                        out_hbm.at[pl.ds(base + c * _G_CH, _G_CH)])


def _gather_feat(feature, row_token_phys):
    out = pl.kernel(
        _gather_feat_kernel,
        mesh=_sc_mesh(),
        out_type=jax.ShapeDtypeStruct((P * _G_SPLIT, _G_W), jnp.float32),
        scratch_types=[
            pltpu.VMEM((_G_CH,), jnp.int32),
            pltpu.VMEM((_G_CH,), jnp.int32),
            pltpu.VMEM((_G_CH,), jnp.int32),
            pltpu.VMEM((_G_CH,), jnp.int32),
            pltpu.VMEM((_G_CH, _G_W), jnp.float32),
            pltpu.VMEM((_G_CH, _G_W), jnp.float32),
            pltpu.SemaphoreType.DMA,
            pltpu.SemaphoreType.DMA,
        ],
    )(feature.reshape(NUM_TOKENS * _G_SPLIT, _G_W), row_token_phys)
    return out.reshape(P, INPUT_DIM)


_C_ROWS = NUM_TOKENS // _NW


def _combine_kernel(slog_hbm, idx_hbm, out_hbm, idx_v, rows_v, sem):
    wid = lax.axis_index("s") * 2 + lax.axis_index("c")
    base = wid * _C_ROWS
    pltpu.sync_copy(idx_hbm.at[pl.ds(base, _C_ROWS)], idx_v)
    pltpu.async_copy(slog_hbm.at[idx_v], rows_v, sem).wait()
    pltpu.sync_copy(rows_v, out_hbm.at[pl.ds(base, _C_ROWS)])


def _combine(sorted_logits, p):
    return pl.kernel(
        _combine_kernel,
        mesh=_sc_mesh(),
        out_type=jax.ShapeDtypeStruct((NUM_TOKENS, NUM_CLASSES), jnp.float32),
        scratch_types=[
            pltpu.VMEM((_C_ROWS,), jnp.int32),
            pltpu.VMEM((_C_ROWS, NUM_CLASSES), jnp.float32),
            pltpu.SemaphoreType.DMA,
        ],
    )(sorted_logits, p)


def _mm_body(nu_ref, bt_ref, x_ref, w_ref, b_ref, o_ref):
    i = pl.program_id(0)

    @pl.when(i < nu_ref[0])
    def _():
        x = x_ref[...]
        w = w_ref[0]
        y = lax.dot_general(x, w, (((1,), (1,)), ((), ())),
                            preferred_element_type=jnp.float32)
        o_ref[...] = y + b_ref[0, 0][None, :]


def _grouped_matmul(sorted_feat, W, b, nb_used, block_task):
    grid_spec = pltpu.PrefetchScalarGridSpec(
        num_scalar_prefetch=2,
        grid=(NB,),
        in_specs=[
            pl.BlockSpec((TB, INPUT_DIM), lambda i, nu, bt: (i, 0)),
            pl.BlockSpec((1, NUM_CLASSES, INPUT_DIM),
                         lambda i, nu, bt: (bt[i], 0, 0)),
            pl.BlockSpec((1, 1, NUM_CLASSES), lambda i, nu, bt: (bt[i], 0, 0)),
        ],
        out_specs=pl.BlockSpec((TB, NUM_CLASSES), lambda i, nu, bt: (i, 0)),
    )
    return pl.pallas_call(
        _mm_body,
        grid_spec=grid_spec,
        out_shape=jax.ShapeDtypeStruct((P, NUM_CLASSES), jnp.float32),
    )(nb_used, block_task, sorted_feat, W,
      b.reshape(NUM_TASKS, 1, NUM_CLASSES))


def kernel(feature, task_ids, W, b):
    t = task_ids.astype(jnp.int32)
    onehot = (t[:, None] == jnp.arange(NUM_TASKS, dtype=jnp.int32)[None, :]
              ).astype(jnp.int32)
    counts = jnp.sum(onehot, axis=0)
    rank = jnp.take_along_axis(jnp.cumsum(onehot, axis=0) - 1,
                               t[:, None], axis=1)[:, 0]
    blocks_per = (counts + TB - 1) // TB
    cumb = jnp.cumsum(blocks_per)
    padded_off = TB * jnp.concatenate(
        [jnp.zeros((1,), jnp.int32), cumb[:-1]])
    p = (padded_off[t] + rank).astype(jnp.int32)
    row_token = (jnp.arange(P, dtype=jnp.int32) % NUM_TOKENS).at[p].set(
        jnp.arange(NUM_TOKENS, dtype=jnp.int32))
    row_token_phys = (_G_SPLIT * row_token[:, None]
                      + jnp.arange(_G_SPLIT, dtype=jnp.int32)[None, :]
                      ).reshape(P * _G_SPLIT)
    nb_used = cumb[-1:].astype(jnp.int32)
    block_task = jnp.minimum(
        jnp.searchsorted(cumb, jnp.arange(NB, dtype=jnp.int32),
                         side="right"),
        NUM_TASKS - 1).astype(jnp.int32)

    sorted_feat = _gather_feat(feature, row_token_phys)
    sorted_logits = _grouped_matmul(sorted_feat, W, b, nb_used, block_task)
    return _combine(sorted_logits, p)


# speedup vs baseline: 1.6699x; 1.2219x over previous
import functools

import jax
import jax.numpy as jnp
from jax import lax
from jax.experimental import pallas as pl
from jax.experimental.pallas import tpu as pltpu
from jax.experimental.pallas import tpu_sc as plsc

NUM_TASKS = 8
NUM_TOKENS = 4096
INPUT_DIM = 1024
NUM_CLASSES = 512

TB = 256
NB = NUM_TOKENS // TB + NUM_TASKS
P = NB * TB

_NW = 32


def _sc_mesh():
    return plsc.VectorSubcoreMesh(core_axis_name="c", subcore_axis_name="s")


_S_W = 512
_S_SPLIT = INPUT_DIM // _S_W
_S_ROWS = NUM_TOKENS * _S_SPLIT // _NW
_S_CH = 64
_S_NCH = _S_ROWS // _S_CH


def _scatter_feat_kernel(feat_hbm, idx_hbm, out_hbm, i0, i1, i2, i3,
                         rows0, rows1, sem0, sem1):
    wid = lax.axis_index("s") * 2 + lax.axis_index("c")
    base = wid * _S_ROWS
    idxs = (i0, i1, i2, i3)
    for c in range(_S_NCH):
        pltpu.sync_copy(idx_hbm.at[pl.ds(base + c * _S_CH, _S_CH)], idxs[c])
    bufs, sems, cps = (rows0, rows1), (sem0, sem1), [None, None]
    for c in range(_S_NCH):
        if c >= 2:
            cps[c % 2].wait()
        pltpu.sync_copy(feat_hbm.at[pl.ds(base + c * _S_CH, _S_CH)],
                        bufs[c % 2])
        cps[c % 2] = pltpu.async_copy(bufs[c % 2], out_hbm.at[idxs[c]],
                                      sems[c % 2])
    cps[0].wait()
    cps[1].wait()


def _scatter_feat(feature, p_phys):
    out = pl.kernel(
        _scatter_feat_kernel,
        mesh=_sc_mesh(),
        out_type=jax.ShapeDtypeStruct((P * _S_SPLIT, _S_W), jnp.float32),
        scratch_types=[
            pltpu.VMEM((_S_CH,), jnp.int32),
            pltpu.VMEM((_S_CH,), jnp.int32),
            pltpu.VMEM((_S_CH,), jnp.int32),
            pltpu.VMEM((_S_CH,), jnp.int32),
            pltpu.VMEM((_S_CH, _S_W), jnp.float32),
            pltpu.VMEM((_S_CH, _S_W), jnp.float32),
            pltpu.SemaphoreType.DMA,
            pltpu.SemaphoreType.DMA,
        ],
    )(feature.reshape(NUM_TOKENS * _S_SPLIT, _S_W), p_phys)
    return out.reshape(P, INPUT_DIM)


_C_ROWS = NUM_TOKENS // _NW


def _combine_kernel(slog_hbm, idx_hbm, out_hbm, idx_v, rows_v, sem):
    wid = lax.axis_index("s") * 2 + lax.axis_index("c")
    base = wid * _C_ROWS
    pltpu.sync_copy(idx_hbm.at[pl.ds(base, _C_ROWS)], idx_v)
    pltpu.async_copy(slog_hbm.at[idx_v], rows_v, sem).wait()
    pltpu.sync_copy(rows_v, out_hbm.at[pl.ds(base, _C_ROWS)])


def _combine(sorted_logits, p):
    return pl.kernel(
        _combine_kernel,
        mesh=_sc_mesh(),
        out_type=jax.ShapeDtypeStruct((NUM_TOKENS, NUM_CLASSES), jnp.float32),
        scratch_types=[
            pltpu.VMEM((_C_ROWS,), jnp.int32),
            pltpu.VMEM((_C_ROWS, NUM_CLASSES), jnp.float32),
            pltpu.SemaphoreType.DMA,
        ],
    )(sorted_logits, p)


def _mm_body(nu_ref, bt_ref, x_ref, w_ref, b_ref, o_ref):
    i = pl.program_id(0)

    @pl.when(i < nu_ref[0])
    def _():
        x = x_ref[...]
        w = w_ref[0]
        y = lax.dot_general(x, w, (((1,), (1,)), ((), ())),
                            preferred_element_type=jnp.float32)
        o_ref[...] = y + b_ref[0, 0][None, :]


def _grouped_matmul(sorted_feat, W, b, nb_used, block_task):
    grid_spec = pltpu.PrefetchScalarGridSpec(
        num_scalar_prefetch=2,
        grid=(NB,),
        in_specs=[
            pl.BlockSpec((TB, INPUT_DIM), lambda i, nu, bt: (i, 0)),
            pl.BlockSpec((1, NUM_CLASSES, INPUT_DIM),
                         lambda i, nu, bt: (bt[i], 0, 0)),
            pl.BlockSpec((1, 1, NUM_CLASSES), lambda i, nu, bt: (bt[i], 0, 0)),
        ],
        out_specs=pl.BlockSpec((TB, NUM_CLASSES), lambda i, nu, bt: (i, 0)),
    )
    return pl.pallas_call(
        _mm_body,
        grid_spec=grid_spec,
        out_shape=jax.ShapeDtypeStruct((P, NUM_CLASSES), jnp.float32),
    )(nb_used, block_task, sorted_feat, W,
      b.reshape(NUM_TASKS, 1, NUM_CLASSES))


def kernel(feature, task_ids, W, b):
    t = task_ids.astype(jnp.int32)
    onehot = (t[:, None] == jnp.arange(NUM_TASKS, dtype=jnp.int32)[None, :]
              ).astype(jnp.int32)
    csum = jnp.cumsum(onehot, axis=0)
    rank = jnp.sum(onehot * (csum - 1), axis=1)
    counts = csum[-1]
    blocks_per = (counts + TB - 1) // TB
    cumb = jnp.cumsum(blocks_per)
    padded_off = TB * (cumb - blocks_per)
    p = (jnp.sum(onehot * padded_off[None, :], axis=1)
         + rank).astype(jnp.int32)
    p_phys = (_S_SPLIT * p[:, None]
              + jnp.arange(_S_SPLIT, dtype=jnp.int32)[None, :]
              ).reshape(NUM_TOKENS * _S_SPLIT)
    nb_used = cumb[-1:].astype(jnp.int32)
    block_task = jnp.minimum(
        jnp.sum((cumb[None, :] <= jnp.arange(NB, dtype=jnp.int32)[:, None]
                 ).astype(jnp.int32), axis=1),
        NUM_TASKS - 1).astype(jnp.int32)

    sorted_feat = _scatter_feat(feature, p_phys)
    sorted_logits = _grouped_matmul(sorted_feat, W, b, nb_used, block_task)
    return _combine(sorted_logits, p)


# speedup vs baseline: 2.7366x; 1.6388x over previous
import functools

import jax
import jax.numpy as jnp
from jax import lax
from jax.experimental import pallas as pl
from jax.experimental.pallas import tpu as pltpu
from jax.experimental.pallas import tpu_sc as plsc

NUM_TASKS = 8
NUM_TOKENS = 4096
INPUT_DIM = 1024
NUM_CLASSES = 512

TB = 256
NB = NUM_TOKENS // TB + NUM_TASKS
P = NB * TB

_NW = 32


def _sc_mesh():
    return plsc.VectorSubcoreMesh(core_axis_name="c", subcore_axis_name="s")


_S_ROWS = NUM_TOKENS // _NW
_S_CH = 32
_S_NCH = _S_ROWS // _S_CH


def _scatter_feat_kernel(feat_hbm, idx_hbm, out_hbm, i0, i1, i2, i3,
                         rows0, rows1, sem0, sem1):
    wid = lax.axis_index("s") * 2 + lax.axis_index("c")
    base = wid * _S_ROWS
    idxs = (i0, i1, i2, i3)
    for c in range(_S_NCH):
        pltpu.sync_copy(idx_hbm.at[pl.ds(base + c * _S_CH, _S_CH)], idxs[c])
    bufs, sems, cps = (rows0, rows1), (sem0, sem1), [None, None]
    for c in range(_S_NCH):
        if c >= 2:
            cps[c % 2].wait()
        pltpu.sync_copy(feat_hbm.at[pl.ds(base + c * _S_CH, _S_CH)],
                        bufs[c % 2])
        cps[c % 2] = pltpu.async_copy(bufs[c % 2], out_hbm.at[idxs[c]],
                                      sems[c % 2])
    cps[0].wait()
    cps[1].wait()


def _scatter_feat(feature, p):
    return pl.kernel(
        _scatter_feat_kernel,
        mesh=_sc_mesh(),
        out_type=jax.ShapeDtypeStruct((P, INPUT_DIM), jnp.float32),
        scratch_types=[
            pltpu.VMEM((_S_CH,), jnp.int32),
            pltpu.VMEM((_S_CH,), jnp.int32),
            pltpu.VMEM((_S_CH,), jnp.int32),
            pltpu.VMEM((_S_CH,), jnp.int32),
            pltpu.VMEM((_S_CH, INPUT_DIM), jnp.float32),
            pltpu.VMEM((_S_CH, INPUT_DIM), jnp.float32),
            pltpu.SemaphoreType.DMA,
            pltpu.SemaphoreType.DMA,
        ],
    )(feature, p)


_C_ROWS = NUM_TOKENS // _NW


def _combine_kernel(slog_hbm, idx_hbm, out_hbm, idx_v, rows_v, sem):
    wid = lax.axis_index("s") * 2 + lax.axis_index("c")
    base = wid * _C_ROWS
    pltpu.sync_copy(idx_hbm.at[pl.ds(base, _C_ROWS)], idx_v)
    pltpu.async_copy(slog_hbm.at[idx_v], rows_v, sem).wait()
    pltpu.sync_copy(rows_v, out_hbm.at[pl.ds(base, _C_ROWS)])


def _combine(sorted_logits, p):
    return pl.kernel(
        _combine_kernel,
        mesh=_sc_mesh(),
        out_type=jax.ShapeDtypeStruct((NUM_TOKENS, NUM_CLASSES), jnp.float32),
        scratch_types=[
            pltpu.VMEM((_C_ROWS,), jnp.int32),
            pltpu.VMEM((_C_ROWS, NUM_CLASSES), jnp.float32),
            pltpu.SemaphoreType.DMA,
        ],
    )(sorted_logits, p)


def _mm_body(nu_ref, bt_ref, x_ref, w_ref, b_ref, o_ref):
    i = pl.program_id(0)

    @pl.when(i < nu_ref[0])
    def _():
        x = x_ref[...]
        w = w_ref[0]
        y = lax.dot_general(x, w, (((1,), (1,)), ((), ())),
                            preferred_element_type=jnp.float32)
        o_ref[...] = y + b_ref[0, 0][None, :]


def _grouped_matmul(sorted_feat, W, b, nb_used, block_task):
    grid_spec = pltpu.PrefetchScalarGridSpec(
        num_scalar_prefetch=2,
        grid=(NB,),
        in_specs=[
            pl.BlockSpec((TB, INPUT_DIM),
                         lambda i, nu, bt: (jnp.minimum(i, nu[0] - 1), 0)),
            pl.BlockSpec((1, NUM_CLASSES, INPUT_DIM),
                         lambda i, nu, bt: (bt[i], 0, 0)),
            pl.BlockSpec((1, 1, NUM_CLASSES), lambda i, nu, bt: (bt[i], 0, 0)),
        ],
        out_specs=pl.BlockSpec((TB, NUM_CLASSES), lambda i, nu, bt: (i, 0)),
    )
    return pl.pallas_call(
        _mm_body,
        grid_spec=grid_spec,
        out_shape=jax.ShapeDtypeStruct((P, NUM_CLASSES), jnp.float32),
    )(nb_used, block_task, sorted_feat, W,
      b.reshape(NUM_TASKS, 1, NUM_CLASSES))


def kernel(feature, task_ids, W, b):
    t = task_ids.astype(jnp.int32)
    onehot = (t[:, None] == jnp.arange(NUM_TASKS, dtype=jnp.int32)[None, :]
              ).astype(jnp.int32)
    csum = jnp.cumsum(onehot, axis=0)
    rank = jnp.sum(onehot * (csum - 1), axis=1)
    counts = csum[-1]
    blocks_per = (counts + TB - 1) // TB
    cumb = jnp.cumsum(blocks_per)
    padded_off = TB * (cumb - blocks_per)
    p = (jnp.sum(onehot * padded_off[None, :], axis=1)
         + rank).astype(jnp.int32)
    nb_used = cumb[-1:].astype(jnp.int32)
    block_task = jnp.minimum(
        jnp.sum((cumb[None, :] <= jnp.arange(NB, dtype=jnp.int32)[:, None]
                 ).astype(jnp.int32), axis=1),
        NUM_TASKS - 1).astype(jnp.int32)

    sorted_feat = _scatter_feat(feature, p)
    sorted_logits = _grouped_matmul(sorted_feat, W, b, nb_used, block_task)
    return _combine(sorted_logits, p)


# speedup vs baseline: 2.8867x; 1.0548x over previous
import functools

import jax
import jax.numpy as jnp
from jax import lax
from jax.experimental import pallas as pl
from jax.experimental.pallas import tpu as pltpu
from jax.experimental.pallas import tpu_sc as plsc

NUM_TASKS = 8
NUM_TOKENS = 4096
INPUT_DIM = 1024
NUM_CLASSES = 512

TB = 512
NB = NUM_TOKENS // TB + NUM_TASKS
P = NB * TB

_NW = 32


def _sc_mesh():
    return plsc.VectorSubcoreMesh(core_axis_name="c", subcore_axis_name="s")


_S_ROWS = NUM_TOKENS // _NW
_S_CH = 32
_S_NCH = _S_ROWS // _S_CH


def _scatter_feat_kernel(feat_hbm, idx_hbm, out_hbm, i0, i1, i2, i3,
                         rows0, rows1, sem0, sem1):
    wid = lax.axis_index("s") * 2 + lax.axis_index("c")
    base = wid * _S_ROWS
    idxs = (i0, i1, i2, i3)
    for c in range(_S_NCH):
        pltpu.sync_copy(idx_hbm.at[pl.ds(base + c * _S_CH, _S_CH)], idxs[c])
    bufs, sems, cps = (rows0, rows1), (sem0, sem1), [None, None]
    for c in range(_S_NCH):
        if c >= 2:
            cps[c % 2].wait()
        pltpu.sync_copy(feat_hbm.at[pl.ds(base + c * _S_CH, _S_CH)],
                        bufs[c % 2])
        cps[c % 2] = pltpu.async_copy(bufs[c % 2], out_hbm.at[idxs[c]],
                                      sems[c % 2])
    cps[0].wait()
    cps[1].wait()


def _scatter_feat(feature, p):
    return pl.kernel(
        _scatter_feat_kernel,
        mesh=_sc_mesh(),
        out_type=jax.ShapeDtypeStruct((P, INPUT_DIM), jnp.float32),
        scratch_types=[
            pltpu.VMEM((_S_CH,), jnp.int32),
            pltpu.VMEM((_S_CH,), jnp.int32),
            pltpu.VMEM((_S_CH,), jnp.int32),
            pltpu.VMEM((_S_CH,), jnp.int32),
            pltpu.VMEM((_S_CH, INPUT_DIM), jnp.float32),
            pltpu.VMEM((_S_CH, INPUT_DIM), jnp.float32),
            pltpu.SemaphoreType.DMA,
            pltpu.SemaphoreType.DMA,
        ],
    )(feature, p)


_C_ROWS = NUM_TOKENS // _NW


def _combine_kernel(slog_hbm, idx_hbm, out_hbm, idx_v, rows_v, sem):
    wid = lax.axis_index("s") * 2 + lax.axis_index("c")
    base = wid * _C_ROWS
    pltpu.sync_copy(idx_hbm.at[pl.ds(base, _C_ROWS)], idx_v)
    pltpu.async_copy(slog_hbm.at[idx_v], rows_v, sem).wait()
    pltpu.sync_copy(rows_v, out_hbm.at[pl.ds(base, _C_ROWS)])


def _combine(sorted_logits, p):
    return pl.kernel(
        _combine_kernel,
        mesh=_sc_mesh(),
        out_type=jax.ShapeDtypeStruct((NUM_TOKENS, NUM_CLASSES), jnp.float32),
        scratch_types=[
            pltpu.VMEM((_C_ROWS,), jnp.int32),
            pltpu.VMEM((_C_ROWS, NUM_CLASSES), jnp.float32),
            pltpu.SemaphoreType.DMA,
        ],
    )(sorted_logits, p)


def _mm_body(nu_ref, bt_ref, x_ref, w_ref, b_ref, o_ref):
    i = pl.program_id(0)

    @pl.when(i < nu_ref[0])
    def _():
        x = x_ref[...]
        w = w_ref[0]
        y = lax.dot_general(x, w, (((1,), (1,)), ((), ())),
                            preferred_element_type=jnp.float32)
        o_ref[...] = y + b_ref[0, 0][None, :]


def _grouped_matmul(sorted_feat, W, b, nb_used, block_task):
    grid_spec = pltpu.PrefetchScalarGridSpec(
        num_scalar_prefetch=2,
        grid=(NB,),
        in_specs=[
            pl.BlockSpec((TB, INPUT_DIM),
                         lambda i, nu, bt: (jnp.minimum(i, nu[0] - 1), 0)),
            pl.BlockSpec((1, NUM_CLASSES, INPUT_DIM),
                         lambda i, nu, bt: (bt[i], 0, 0)),
            pl.BlockSpec((1, 1, NUM_CLASSES), lambda i, nu, bt: (bt[i], 0, 0)),
        ],
        out_specs=pl.BlockSpec((TB, NUM_CLASSES),
                               lambda i, nu, bt: (jnp.minimum(i, nu[0] - 1),
                                                  0)),
    )
    return pl.pallas_call(
        _mm_body,
        grid_spec=grid_spec,
        out_shape=jax.ShapeDtypeStruct((P, NUM_CLASSES), jnp.float32),
    )(nb_used, block_task, sorted_feat, W,
      b.reshape(NUM_TASKS, 1, NUM_CLASSES))


def kernel(feature, task_ids, W, b):
    t = task_ids.astype(jnp.int32)
    onehot = (t[:, None] == jnp.arange(NUM_TASKS, dtype=jnp.int32)[None, :]
              ).astype(jnp.int32)
    csum = jnp.cumsum(onehot, axis=0)
    rank = jnp.sum(onehot * (csum - 1), axis=1)
    counts = csum[-1]
    blocks_per = (counts + TB - 1) // TB
    cumb = jnp.cumsum(blocks_per)
    padded_off = TB * (cumb - blocks_per)
    p = (jnp.sum(onehot * padded_off[None, :], axis=1)
         + rank).astype(jnp.int32)
    nb_used = cumb[-1:].astype(jnp.int32)
    block_task = jnp.minimum(
        jnp.sum((cumb[None, :] <= jnp.arange(NB, dtype=jnp.int32)[:, None]
                 ).astype(jnp.int32), axis=1),
        NUM_TASKS - 1).astype(jnp.int32)

    sorted_feat = _scatter_feat(feature, p)
    sorted_logits = _grouped_matmul(sorted_feat, W, b, nb_used, block_task)
    return _combine(sorted_logits, p)
